# baseline (speedup 1.0000x reference)
import functools

import jax
import jax.numpy as jnp
from jax import lax
from jax.experimental import pallas as pl
from jax.experimental.pallas import tpu as pltpu
from jax.experimental.pallas import tpu_sc as plsc

DIM = 768
HID = 3072
NE = 8
N = 2048
BLK = 128
P = N + NE * BLK
NB = P // BLK

NC = 2
NS = 16
NW = NC * NS
TPW = N // NW


def _plan_body(x_ref, rw_ref, rb_ref, pos_ref, be_ref, slot_ref, new_ref,
               fn_ref, ne_ref, live_ref):
    x = x_ref[...]
    logits = jnp.dot(x, rw_ref[...],
                     preferred_element_type=jnp.float32) + rb_ref[...]
    mx = jnp.max(logits, axis=1, keepdims=True)
    e = jnp.exp(logits - mx)
    w = e / jnp.sum(e, axis=1, keepdims=True)
    wmx = jnp.max(w, axis=1, keepdims=True)
    eids = lax.broadcasted_iota(jnp.int32, (N, NE), 1)
    best = jnp.min(jnp.where(w >= wmx, eids, NE), axis=1,
                   keepdims=True)
    onehot = (eids == best).astype(jnp.float32)

    counts = jnp.sum(onehot, axis=0, keepdims=True)
    counts_i = counts.astype(jnp.int32)
    padded = ((counts_i + (BLK - 1)) // BLK) * BLK
    padded_f = padded.astype(jnp.float32)
    er = lax.broadcasted_iota(jnp.int32, (NE, NE), 0)
    ec = lax.broadcasted_iota(jnp.int32, (NE, NE), 1)
    tri = (er < ec).astype(jnp.float32)
    starts = jnp.dot(padded_f, tri,
                     preferred_element_type=jnp.float32)

    CH = 256
    ci = lax.broadcasted_iota(jnp.int32, (CH, CH), 0)
    cj = lax.broadcasted_iota(jnp.int32, (CH, CH), 1)
    ltri = (cj <= ci).astype(jnp.float32)
    carry = jnp.zeros((1, NE), jnp.float32)
    ranks = []
    for c in range(N // CH):
        blk = onehot[c * CH:(c + 1) * CH]
        incl_c = jnp.dot(ltri, blk,
                         preferred_element_type=jnp.float32) + carry
        ranks.append(jnp.sum(incl_c * blk, axis=1, keepdims=True))
        carry = carry + jnp.sum(blk, axis=0, keepdims=True)
    rank = jnp.concatenate(ranks, axis=0) - 1.0
    start_tok = jnp.sum(onehot * starts, axis=1, keepdims=True)
    pos_ref[...] = (start_tok + rank).astype(jnp.int32)

    ends = starts + padded_f
    ib = (lax.broadcasted_iota(jnp.int32, (NB, NE), 0) * BLK).astype(
        jnp.float32)
    be = jnp.sum((ib >= ends).astype(jnp.int32), axis=1, keepdims=True)
    be_c = jnp.minimum(be, NE - 1)
    be_ref[...] = be_c

    prev = jnp.concatenate(
        [jnp.full((1, 1), -1, jnp.int32), be_c[:-1]], axis=0)
    newrun = (be_c != prev).astype(jnp.int32)
    bi = lax.broadcasted_iota(jnp.int32, (NB, NB), 0)
    bj = lax.broadcasted_iota(jnp.int32, (NB, NB), 1)
    ltri_b = (bj <= bi).astype(jnp.float32)
    runidx = jnp.dot(ltri_b, newrun.astype(jnp.float32),
                     preferred_element_type=jnp.float32).astype(jnp.int32) - 1
    slot_ref[...] = runidx % 2
    new_ref[...] = newrun
    eids8 = lax.broadcasted_iota(jnp.int32, (NB, NE), 1)
    pres = jnp.max((eids8 == be_c).astype(jnp.int32), axis=0,
                   keepdims=True)
    cand = jnp.where((eids8 > be_c) & (pres > 0), eids8, 99)
    nxt = jnp.min(cand, axis=1, keepdims=True)
    fn_ref[...] = (nxt < 99).astype(jnp.int32)
    ne_ref[...] = jnp.minimum(nxt, NE - 1)
    total = jnp.sum(padded)
    blk_start = lax.broadcasted_iota(jnp.int32, (NB, 1), 0) * BLK
    live_ref[...] = (blk_start < total).astype(jnp.int32)


def _plan(x, router_W, router_b):
    return pl.pallas_call(
        _plan_body,
        out_shape=(jax.ShapeDtypeStruct((N, 1), jnp.int32),)
        + tuple(jax.ShapeDtypeStruct((NB, 1), jnp.int32) for _ in range(6)),
    )(x, router_W, router_b.reshape(1, NE))


def _dispatch_body(x_hbm, pos_hbm, xs_hbm, idx_v, rows_v, sem):
    wid = lax.axis_index("s") * NC + lax.axis_index("c")
    base = wid * TPW
    pltpu.sync_copy(pos_hbm.at[pl.ds(base, TPW)], idx_v)
    pltpu.sync_copy(x_hbm.at[pl.ds(base, TPW)], rows_v)
    pltpu.async_copy(rows_v, xs_hbm.at[idx_v], sem).wait()


def _combine_body(ys_hbm, pos_hbm, out_hbm, idx_v, rows_v, sem):
    wid = lax.axis_index("s") * NC + lax.axis_index("c")
    base = wid * TPW
    pltpu.sync_copy(pos_hbm.at[pl.ds(base, TPW)], idx_v)
    pltpu.async_copy(ys_hbm.at[idx_v], rows_v, sem).wait()
    pltpu.sync_copy(rows_v, out_hbm.at[pl.ds(base, TPW)])


@functools.lru_cache(maxsize=None)
def _sc_kernels():
    mesh = plsc.VectorSubcoreMesh(core_axis_name="c", subcore_axis_name="s")
    scratch = [pltpu.VMEM((TPW,), jnp.int32),
               pltpu.VMEM((TPW, DIM), jnp.float32),
               pltpu.SemaphoreType.DMA]
    dispatch = pl.kernel(
        _dispatch_body, mesh=mesh,
        out_type=jax.ShapeDtypeStruct((P, DIM), jnp.float32),
        scratch_types=scratch)
    combine = pl.kernel(
        _combine_body, mesh=mesh,
        out_type=jax.ShapeDtypeStruct((N, DIM), jnp.float32),
        scratch_types=scratch)
    return dispatch, combine


def _ffn_body(be_ref, slot_ref, new_ref, fn_ref, ne_ref, live_ref,
              x_ref, w1_hbm, b1_ref, w2_hbm, b2_ref, y_ref,
              w1_buf, w2_buf, sems):
    i = pl.program_id(0)
    cure = be_ref[i, 0]
    slot = slot_ref[i, 0]

    H2 = DIM // 2

    def start(e, s):
        pltpu.make_async_copy(w1_hbm.at[e, pl.ds(0, H2)],
                              w1_buf.at[s, pl.ds(0, H2)],
                              sems.at[s, 0]).start()
        pltpu.make_async_copy(w1_hbm.at[e, pl.ds(H2, H2)],
                              w1_buf.at[s, pl.ds(H2, H2)],
                              sems.at[s, 1]).start()
        pltpu.make_async_copy(w2_hbm.at[e, pl.ds(0, HID // 2)],
                              w2_buf.at[s, pl.ds(0, HID // 2)],
                              sems.at[s, 2]).start()
        pltpu.make_async_copy(w2_hbm.at[e, pl.ds(HID // 2, HID // 2)],
                              w2_buf.at[s, pl.ds(HID // 2, HID // 2)],
                              sems.at[s, 3]).start()

    @pl.when(i == 0)
    def _():
        start(cure, slot)

    @pl.when((new_ref[i, 0] == 1) & (fn_ref[i, 0] == 1))
    def _():
        start(ne_ref[i, 0], 1 - slot)

    @pl.when(new_ref[i, 0] == 1)
    def _():
        pltpu.make_async_copy(w1_hbm.at[cure, pl.ds(0, H2)],
                              w1_buf.at[slot, pl.ds(0, H2)],
                              sems.at[slot, 0]).wait()
        pltpu.make_async_copy(w1_hbm.at[cure, pl.ds(H2, H2)],
                              w1_buf.at[slot, pl.ds(H2, H2)],
                              sems.at[slot, 1]).wait()
        pltpu.make_async_copy(w2_hbm.at[cure, pl.ds(0, HID // 2)],
                              w2_buf.at[slot, pl.ds(0, HID // 2)],
                              sems.at[slot, 2]).wait()
        pltpu.make_async_copy(w2_hbm.at[cure, pl.ds(HID // 2, HID // 2)],
                              w2_buf.at[slot, pl.ds(HID // 2, HID // 2)],
                              sems.at[slot, 3]).wait()

    live = live_ref[i, 0] == 1

    @pl.when(live & (slot == 0))
    def _():
        h = jnp.maximum(jnp.dot(x_ref[...], w1_buf[0],
                                preferred_element_type=jnp.float32)
                        + b1_ref[0], 0.0)
        y_ref[...] = jnp.dot(h, w2_buf[0],
                             preferred_element_type=jnp.float32) + b2_ref[0]

    @pl.when(live & (slot == 1))
    def _():
        h = jnp.maximum(jnp.dot(x_ref[...], w1_buf[1],
                                preferred_element_type=jnp.float32)
                        + b1_ref[0], 0.0)
        y_ref[...] = jnp.dot(h, w2_buf[1],
                             preferred_element_type=jnp.float32) + b2_ref[0]


def _ffn(sched, xs, W1, b1, W2, b2):
    grid_spec = pltpu.PrefetchScalarGridSpec(
        num_scalar_prefetch=6,
        grid=(NB,),
        in_specs=[
            pl.BlockSpec((BLK, DIM), lambda i, be, sl, nw, fn, ne, lv: (i, 0)),
            pl.BlockSpec(memory_space=pl.ANY),
            pl.BlockSpec((1, 1, HID),
                         lambda i, be, sl, nw, fn, ne, lv: (be[i, 0], 0, 0)),
            pl.BlockSpec(memory_space=pl.ANY),
            pl.BlockSpec((1, 1, DIM),
                         lambda i, be, sl, nw, fn, ne, lv: (be[i, 0], 0, 0)),
        ],
        out_specs=pl.BlockSpec((BLK, DIM),
                               lambda i, be, sl, nw, fn, ne, lv: (i, 0)),
        scratch_shapes=[
            pltpu.VMEM((2, DIM, HID), jnp.float32),
            pltpu.VMEM((2, HID, DIM), jnp.float32),
            pltpu.SemaphoreType.DMA((2, 4)),
        ],
    )
    return pl.pallas_call(
        _ffn_body,
        grid_spec=grid_spec,
        out_shape=jax.ShapeDtypeStruct((P, DIM), jnp.float32),
        compiler_params=pltpu.CompilerParams(
            dimension_semantics=("arbitrary",)),
    )(*sched, xs, W1, b1.reshape(NE, 1, HID), W2, b2.reshape(NE, 1, DIM))


def kernel(x, router_W, router_b, W1, b1, W2, b2):
    pos2, *sched = _plan(x, router_W, router_b)
    pos = pos2.reshape(N)
    dispatch, combine = _sc_kernels()
    xs = dispatch(x, pos)
    ys = _ffn(sched, xs, W1, b1, W2, b2)
    return combine(ys, pos)


# speedup vs baseline: 1.0573x; 1.0573x over previous
import functools

import jax
import jax.numpy as jnp
from jax import lax
from jax.experimental import pallas as pl
from jax.experimental.pallas import tpu as pltpu
from jax.experimental.pallas import tpu_sc as plsc

DIM = 768
HID = 3072
NE = 8
N = 2048
BLK = 128
P = N + NE * BLK
NB = P // BLK

NC = 2
NS = 16
NW = NC * NS
TPW = N // NW


def _plan_body(x_ref, rw_ref, rb_ref, pos_ref, be_ref, slot_ref, new_ref,
               fn_ref, ne_ref, live_ref):
    x = x_ref[...]
    logits = jnp.dot(x, rw_ref[...],
                     preferred_element_type=jnp.float32) + rb_ref[...]
    mx = jnp.max(logits, axis=1, keepdims=True)
    e = jnp.exp(logits - mx)
    w = e / jnp.sum(e, axis=1, keepdims=True)
    wmx = jnp.max(w, axis=1, keepdims=True)
    eids = lax.broadcasted_iota(jnp.int32, (N, NE), 1)
    best = jnp.min(jnp.where(w >= wmx, eids, NE), axis=1,
                   keepdims=True)
    onehot = (eids == best).astype(jnp.float32)

    counts = jnp.sum(onehot, axis=0, keepdims=True)
    counts_i = counts.astype(jnp.int32)
    padded = ((counts_i + (BLK - 1)) // BLK) * BLK
    padded_f = padded.astype(jnp.float32)
    er = lax.broadcasted_iota(jnp.int32, (NE, NE), 0)
    ec = lax.broadcasted_iota(jnp.int32, (NE, NE), 1)
    tri = (er < ec).astype(jnp.float32)
    starts = jnp.dot(padded_f, tri,
                     preferred_element_type=jnp.float32)

    CH = 256
    ci = lax.broadcasted_iota(jnp.int32, (CH, CH), 0)
    cj = lax.broadcasted_iota(jnp.int32, (CH, CH), 1)
    ltri = (cj <= ci).astype(jnp.float32)
    carry = jnp.zeros((1, NE), jnp.float32)
    ranks = []
    for c in range(N // CH):
        blk = onehot[c * CH:(c + 1) * CH]
        incl_c = jnp.dot(ltri, blk,
                         preferred_element_type=jnp.float32) + carry
        ranks.append(jnp.sum(incl_c * blk, axis=1, keepdims=True))
        carry = carry + jnp.sum(blk, axis=0, keepdims=True)
    rank = jnp.concatenate(ranks, axis=0) - 1.0
    start_tok = jnp.sum(onehot * starts, axis=1, keepdims=True)
    pos_ref[...] = (start_tok + rank).astype(jnp.int32)

    ends = starts + padded_f
    ib = (lax.broadcasted_iota(jnp.int32, (NB, NE), 0) * BLK).astype(
        jnp.float32)
    be = jnp.sum((ib >= ends).astype(jnp.int32), axis=1, keepdims=True)
    be_c = jnp.minimum(be, NE - 1)
    be_ref[...] = be_c

    prev = jnp.concatenate(
        [jnp.full((1, 1), -1, jnp.int32), be_c[:-1]], axis=0)
    newrun = (be_c != prev).astype(jnp.int32)
    bi = lax.broadcasted_iota(jnp.int32, (NB, NB), 0)
    bj = lax.broadcasted_iota(jnp.int32, (NB, NB), 1)
    ltri_b = (bj <= bi).astype(jnp.float32)
    runidx = jnp.dot(ltri_b, newrun.astype(jnp.float32),
                     preferred_element_type=jnp.float32).astype(jnp.int32) - 1
    slot_ref[...] = runidx % 2
    new_ref[...] = newrun
    eids8 = lax.broadcasted_iota(jnp.int32, (NB, NE), 1)
    pres = jnp.max((eids8 == be_c).astype(jnp.int32), axis=0,
                   keepdims=True)
    cand = jnp.where((eids8 > be_c) & (pres > 0), eids8, 99)
    nxt = jnp.min(cand, axis=1, keepdims=True)
    fn_ref[...] = (nxt < 99).astype(jnp.int32)
    ne_ref[...] = jnp.minimum(nxt, NE - 1)
    total = jnp.sum(padded)
    blk_start = lax.broadcasted_iota(jnp.int32, (NB, 1), 0) * BLK
    live_ref[...] = (blk_start < total).astype(jnp.int32)


def _plan(x, router_W, router_b):
    return pl.pallas_call(
        _plan_body,
        out_shape=(jax.ShapeDtypeStruct((N, 1), jnp.int32),)
        + tuple(jax.ShapeDtypeStruct((NB, 1), jnp.int32) for _ in range(6)),
    )(x, router_W, router_b.reshape(1, NE))


def _combine_body(ys_hbm, pos_hbm, out_hbm, idx_v, rows_v, sem):
    wid = lax.axis_index("s") * NC + lax.axis_index("c")
    base = wid * TPW
    pltpu.sync_copy(pos_hbm.at[pl.ds(base, TPW)], idx_v)
    pltpu.async_copy(ys_hbm.at[idx_v], rows_v, sem).wait()
    pltpu.sync_copy(rows_v, out_hbm.at[pl.ds(base, TPW)])


@functools.lru_cache(maxsize=None)
def _sc_kernels():
    mesh = plsc.VectorSubcoreMesh(core_axis_name="c", subcore_axis_name="s")
    scratch = [pltpu.VMEM((TPW,), jnp.int32),
               pltpu.VMEM((TPW, DIM), jnp.float32),
               pltpu.SemaphoreType.DMA]
    combine = pl.kernel(
        _combine_body, mesh=mesh,
        out_type=jax.ShapeDtypeStruct((N, DIM), jnp.float32),
        scratch_types=scratch)
    return combine


def _ffn_body(be_ref, slot_ref, new_ref, fn_ref, ne_ref, live_ref,
              pos_ref, x_ref, w1_hbm, b1_ref, w2_hbm, b2_ref, y_ref,
              w1_buf, w2_buf, sems):
    i = pl.program_id(0)
    cure = be_ref[i, 0]
    slot = slot_ref[i, 0]

    H2 = DIM // 2

    def start(e, s):
        pltpu.make_async_copy(w1_hbm.at[e, pl.ds(0, H2)],
                              w1_buf.at[s, pl.ds(0, H2)],
                              sems.at[s, 0]).start()
        pltpu.make_async_copy(w1_hbm.at[e, pl.ds(H2, H2)],
                              w1_buf.at[s, pl.ds(H2, H2)],
                              sems.at[s, 1]).start()
        pltpu.make_async_copy(w2_hbm.at[e, pl.ds(0, HID // 2)],
                              w2_buf.at[s, pl.ds(0, HID // 2)],
                              sems.at[s, 2]).start()
        pltpu.make_async_copy(w2_hbm.at[e, pl.ds(HID // 2, HID // 2)],
                              w2_buf.at[s, pl.ds(HID // 2, HID // 2)],
                              sems.at[s, 3]).start()

    @pl.when(i == 0)
    def _():
        start(cure, slot)

    @pl.when((new_ref[i, 0] == 1) & (fn_ref[i, 0] == 1))
    def _():
        start(ne_ref[i, 0], 1 - slot)

    @pl.when(new_ref[i, 0] == 1)
    def _():
        pltpu.make_async_copy(w1_hbm.at[cure, pl.ds(0, H2)],
                              w1_buf.at[slot, pl.ds(0, H2)],
                              sems.at[slot, 0]).wait()
        pltpu.make_async_copy(w1_hbm.at[cure, pl.ds(H2, H2)],
                              w1_buf.at[slot, pl.ds(H2, H2)],
                              sems.at[slot, 1]).wait()
        pltpu.make_async_copy(w2_hbm.at[cure, pl.ds(0, HID // 2)],
                              w2_buf.at[slot, pl.ds(0, HID // 2)],
                              sems.at[slot, 2]).wait()
        pltpu.make_async_copy(w2_hbm.at[cure, pl.ds(HID // 2, HID // 2)],
                              w2_buf.at[slot, pl.ds(HID // 2, HID // 2)],
                              sems.at[slot, 3]).wait()

    live = live_ref[i, 0] == 1

    def compute(s):
        slots = lax.broadcasted_iota(jnp.int32, (BLK, N), 0) + i * BLK
        D = (pos_ref[...] == slots).astype(jnp.float32)
        xb = jnp.dot(D, x_ref[...], preferred_element_type=jnp.float32)
        h = jnp.maximum(jnp.dot(xb, w1_buf[s],
                                preferred_element_type=jnp.float32)
                        + b1_ref[0], 0.0)
        y_ref[...] = jnp.dot(h, w2_buf[s],
                             preferred_element_type=jnp.float32) + b2_ref[0]

    @pl.when(live & (slot == 0))
    def _():
        compute(0)

    @pl.when(live & (slot == 1))
    def _():
        compute(1)


def _ffn(sched, pos_row, x, W1, b1, W2, b2):
    grid_spec = pltpu.PrefetchScalarGridSpec(
        num_scalar_prefetch=6,
        grid=(NB,),
        in_specs=[
            pl.BlockSpec((1, N), lambda i, be, sl, nw, fn, ne, lv: (0, 0)),
            pl.BlockSpec((N, DIM), lambda i, be, sl, nw, fn, ne, lv: (0, 0)),
            pl.BlockSpec(memory_space=pl.ANY),
            pl.BlockSpec((1, 1, HID),
                         lambda i, be, sl, nw, fn, ne, lv: (be[i, 0], 0, 0)),
            pl.BlockSpec(memory_space=pl.ANY),
            pl.BlockSpec((1, 1, DIM),
                         lambda i, be, sl, nw, fn, ne, lv: (be[i, 0], 0, 0)),
        ],
        out_specs=pl.BlockSpec((BLK, DIM),
                               lambda i, be, sl, nw, fn, ne, lv: (i, 0)),
        scratch_shapes=[
            pltpu.VMEM((2, DIM, HID), jnp.float32),
            pltpu.VMEM((2, HID, DIM), jnp.float32),
            pltpu.SemaphoreType.DMA((2, 4)),
        ],
    )
    return pl.pallas_call(
        _ffn_body,
        grid_spec=grid_spec,
        out_shape=jax.ShapeDtypeStruct((P, DIM), jnp.float32),
        compiler_params=pltpu.CompilerParams(
            dimension_semantics=("arbitrary",)),
    )(*sched, pos_row, x, W1, b1.reshape(NE, 1, HID), W2,
      b2.reshape(NE, 1, DIM))


def kernel(x, router_W, router_b, W1, b1, W2, b2):
    pos2, *sched = _plan(x, router_W, router_b)
    combine = _sc_kernels()
    ys = _ffn(sched, pos2.reshape(1, N), x, W1, b1, W2, b2)
    return combine(ys, pos2.reshape(N))
